# baseline (speedup 1.0000x reference)
import functools

import jax
import jax.numpy as jnp
from jax import lax
from jax.experimental import pallas as pl
from jax.experimental.pallas import tpu as pltpu
from jax.experimental.pallas import tpu_sc as plsc

M = 9216
K = 256
N = 8192
BM = 256
BN = 8192
NI = M // BM
NJ = N // BN

_NC = 2
_NS = 16
_NW = _NC * _NS
_BPW = M // _NW
_CH = 96


def _dist_body(x_ref, w_ref, idx_ref, bv_ref, bi_ref):
    j = pl.program_id(1)
    xb = x_ref[...]
    wb = w_ref[...]
    a = (2.0 * xb).astype(jnp.bfloat16)
    dot = jnp.dot(a, wb.astype(jnp.bfloat16),
                  preferred_element_type=jnp.float32)
    xsq = jnp.sum(xb ** 2, axis=1, keepdims=True)
    wsq = jnp.sum(wb ** 2, axis=0, keepdims=True)
    d = (xsq - dot) + wsq
    lmin = jnp.min(d, axis=1, keepdims=True)
    ii = lax.broadcasted_iota(jnp.int32, d.shape, 1) + j * BN
    larg = jnp.min(jnp.where(d == lmin, ii, jnp.int32(2**31 - 1)),
                   axis=1, keepdims=True)

    @pl.when(j == 0)
    def _():
        bv_ref[...] = lmin
        bi_ref[...] = larg

    @pl.when(j > 0)
    def _():
        bv = bv_ref[...]
        upd = lmin < bv
        bv_ref[...] = jnp.where(upd, lmin, bv)
        bi_ref[...] = jnp.where(upd, larg, bi_ref[...])

    @pl.when(j == NJ - 1)
    def _():
        idx_ref[...] = bi_ref[...]


def _argmin_indices(flat, w):
    return pl.pallas_call(
        _dist_body,
        grid=(NI, NJ),
        in_specs=[
            pl.BlockSpec((BM, K), lambda i, j: (i, 0)),
            pl.BlockSpec((K, BN), lambda i, j: (0, j)),
        ],
        out_specs=pl.BlockSpec((BM, 1), lambda i, j: (i, 0)),
        out_shape=jax.ShapeDtypeStruct((M, 1), jnp.int32),
        scratch_shapes=[
            pltpu.VMEM((BM, 1), jnp.float32),
            pltpu.VMEM((BM, 1), jnp.int32),
        ],
        compiler_params=pltpu.CompilerParams(
            dimension_semantics=("parallel", "arbitrary")),
    )(flat, w)


def _tr_body(w_ref, wt_ref):
    wt_ref[...] = w_ref[...].T


def _transpose_w(w):
    return pl.pallas_call(
        _tr_body,
        grid=(NJ,),
        in_specs=[pl.BlockSpec((K, BN), lambda j: (0, j))],
        out_specs=pl.BlockSpec((BN, K), lambda j: (j, 0)),
        out_shape=jax.ShapeDtypeStruct((N, K), jnp.float32),
    )(w)


def _gather_rows(table, idx):
    mesh = plsc.VectorSubcoreMesh(core_axis_name="c", subcore_axis_name="s")

    @functools.partial(
        pl.kernel,
        mesh=mesh,
        out_type=jax.ShapeDtypeStruct((M, K), jnp.float32),
        scratch_types=[
            pltpu.VMEM((_CH,), jnp.int32),
            pltpu.VMEM((_BPW, K), jnp.float32),
            pltpu.SemaphoreType.DMA,
        ],
    )
    def gather_k(table_hbm, idx_hbm, out_hbm, idx_v, rows_v, sem):
        wid = lax.axis_index("s") * _NC + lax.axis_index("c")
        base = wid * _BPW
        for c in range(_BPW // _CH):
            pltpu.sync_copy(idx_hbm.at[pl.ds(base + c * _CH, _CH)], idx_v)
            pltpu.async_copy(table_hbm.at[idx_v],
                             rows_v.at[pl.ds(c * _CH, _CH)], sem).wait()
        pltpu.sync_copy(rows_v, out_hbm.at[pl.ds(base, _BPW)])

    return gather_k(table, idx)


def kernel(x, w):
    flat = x.reshape(-1, K)
    idx2d = _argmin_indices(flat, w)
    wt = _transpose_w(w)
    quant = _gather_rows(wt, idx2d.reshape(M))
    return quant.reshape(x.shape)


# speedup vs baseline: 1.2618x; 1.2618x over previous
import functools

import jax
import jax.numpy as jnp
from jax import lax
from jax.experimental import pallas as pl
from jax.experimental.pallas import tpu as pltpu
from jax.experimental.pallas import tpu_sc as plsc

M = 9216
K = 256
N = 8192
BM = 1024
BN = 8192
NI = M // BM
NJ = N // BN

_NC = 2
_NS = 16
_NW = _NC * _NS
_BPW = M // _NW
_CH = 96


def _dist_body(x_ref, w_ref, idx_ref, bv_ref, bi_ref):
    j = pl.program_id(1)
    xb = x_ref[...]
    wb = w_ref[...]
    a = (2.0 * xb).astype(jnp.bfloat16)
    dot = jnp.dot(a, wb.astype(jnp.bfloat16),
                  preferred_element_type=jnp.float32)
    xsq = jnp.sum(xb ** 2, axis=1, keepdims=True)
    wsq = jnp.sum(wb ** 2, axis=0, keepdims=True)
    d = (xsq - dot) + wsq
    lmin = jnp.min(d, axis=1, keepdims=True)
    ii = lax.broadcasted_iota(jnp.int32, d.shape, 1) + j * BN
    larg = jnp.min(jnp.where(d == lmin, ii, jnp.int32(2**31 - 1)),
                   axis=1, keepdims=True)

    @pl.when(j == 0)
    def _():
        bv_ref[...] = lmin
        bi_ref[...] = larg

    @pl.when(j > 0)
    def _():
        bv = bv_ref[...]
        upd = lmin < bv
        bv_ref[...] = jnp.where(upd, lmin, bv)
        bi_ref[...] = jnp.where(upd, larg, bi_ref[...])

    @pl.when(j == NJ - 1)
    def _():
        idx_ref[...] = bi_ref[...]


def _argmin_indices(flat, w):
    return pl.pallas_call(
        _dist_body,
        grid=(NI, NJ),
        in_specs=[
            pl.BlockSpec((BM, K), lambda i, j: (i, 0)),
            pl.BlockSpec((K, BN), lambda i, j: (0, j)),
        ],
        out_specs=pl.BlockSpec((BM, 1), lambda i, j: (i, 0)),
        out_shape=jax.ShapeDtypeStruct((M, 1), jnp.int32),
        scratch_shapes=[
            pltpu.VMEM((BM, 1), jnp.float32),
            pltpu.VMEM((BM, 1), jnp.int32),
        ],
        compiler_params=pltpu.CompilerParams(
            dimension_semantics=("parallel", "arbitrary")),
    )(flat, w)


def _tr_body(w_ref, wt_ref):
    wt_ref[...] = w_ref[...].T


def _transpose_w(w):
    return pl.pallas_call(
        _tr_body,
        grid=(NJ,),
        in_specs=[pl.BlockSpec((K, BN), lambda j: (0, j))],
        out_specs=pl.BlockSpec((BN, K), lambda j: (j, 0)),
        out_shape=jax.ShapeDtypeStruct((N, K), jnp.float32),
    )(w)


def _gather_rows(table, idx):
    mesh = plsc.VectorSubcoreMesh(core_axis_name="c", subcore_axis_name="s")

    @functools.partial(
        pl.kernel,
        mesh=mesh,
        out_type=jax.ShapeDtypeStruct((M, K), jnp.float32),
        scratch_types=[
            pltpu.VMEM((_CH,), jnp.int32),
            pltpu.VMEM((_BPW, K), jnp.float32),
            pltpu.SemaphoreType.DMA,
        ],
    )
    def gather_k(table_hbm, idx_hbm, out_hbm, idx_v, rows_v, sem):
        wid = lax.axis_index("s") * _NC + lax.axis_index("c")
        base = wid * _BPW
        for c in range(_BPW // _CH):
            pltpu.sync_copy(idx_hbm.at[pl.ds(base + c * _CH, _CH)], idx_v)
            pltpu.async_copy(table_hbm.at[idx_v],
                             rows_v.at[pl.ds(c * _CH, _CH)], sem).wait()
        pltpu.sync_copy(rows_v, out_hbm.at[pl.ds(base, _BPW)])

    return gather_k(table, idx)


def kernel(x, w):
    flat = x.reshape(-1, K)
    idx2d = _argmin_indices(flat, w)
    wt = _transpose_w(w)
    quant = _gather_rows(wt, idx2d.reshape(M))
    return quant.reshape(x.shape)


# speedup vs baseline: 1.2667x; 1.0039x over previous
import functools

import jax
import jax.numpy as jnp
from jax import lax
from jax.experimental import pallas as pl
from jax.experimental.pallas import tpu as pltpu
from jax.experimental.pallas import tpu_sc as plsc

M = 9216
K = 256
N = 8192
BM = 1152
BN = 8192
NI = M // BM
NJ = N // BN

_NC = 2
_NS = 16
_NW = _NC * _NS
_BPW = M // _NW
_CH = 96


def _dist_body(x_ref, w_ref, idx_ref, bv_ref, bi_ref):
    j = pl.program_id(1)
    xb = x_ref[...]
    wb = w_ref[...]
    a = (2.0 * xb).astype(jnp.bfloat16)
    dot = jnp.dot(a, wb.astype(jnp.bfloat16),
                  preferred_element_type=jnp.float32)
    xsq = jnp.sum(xb ** 2, axis=1, keepdims=True)
    wsq = jnp.sum(wb ** 2, axis=0, keepdims=True)
    d = (xsq - dot) + wsq
    lmin = jnp.min(d, axis=1, keepdims=True)
    ii = lax.broadcasted_iota(jnp.int32, d.shape, 1) + j * BN
    larg = jnp.min(jnp.where(d == lmin, ii, jnp.int32(2**31 - 1)),
                   axis=1, keepdims=True)

    @pl.when(j == 0)
    def _():
        bv_ref[...] = lmin
        bi_ref[...] = larg

    @pl.when(j > 0)
    def _():
        bv = bv_ref[...]
        upd = lmin < bv
        bv_ref[...] = jnp.where(upd, lmin, bv)
        bi_ref[...] = jnp.where(upd, larg, bi_ref[...])

    @pl.when(j == NJ - 1)
    def _():
        idx_ref[...] = bi_ref[...]


def _argmin_indices(flat, w):
    return pl.pallas_call(
        _dist_body,
        grid=(NI, NJ),
        in_specs=[
            pl.BlockSpec((BM, K), lambda i, j: (i, 0)),
            pl.BlockSpec((K, BN), lambda i, j: (0, j)),
        ],
        out_specs=pl.BlockSpec((BM, 1), lambda i, j: (i, 0)),
        out_shape=jax.ShapeDtypeStruct((M, 1), jnp.int32),
        scratch_shapes=[
            pltpu.VMEM((BM, 1), jnp.float32),
            pltpu.VMEM((BM, 1), jnp.int32),
        ],
        compiler_params=pltpu.CompilerParams(
            dimension_semantics=("parallel", "arbitrary")),
    )(flat, w)


def _tr_body(w_ref, wt_ref):
    wt_ref[...] = w_ref[...].T


def _transpose_w(w):
    return pl.pallas_call(
        _tr_body,
        grid=(NJ,),
        in_specs=[pl.BlockSpec((K, BN), lambda j: (0, j))],
        out_specs=pl.BlockSpec((BN, K), lambda j: (j, 0)),
        out_shape=jax.ShapeDtypeStruct((N, K), jnp.float32),
    )(w)


def _gather_rows(table, idx):
    mesh = plsc.VectorSubcoreMesh(core_axis_name="c", subcore_axis_name="s")

    @functools.partial(
        pl.kernel,
        mesh=mesh,
        out_type=jax.ShapeDtypeStruct((M, K), jnp.float32),
        scratch_types=[
            pltpu.VMEM((_CH,), jnp.int32),
            pltpu.VMEM((_BPW, K), jnp.float32),
            pltpu.SemaphoreType.DMA,
        ],
    )
    def gather_k(table_hbm, idx_hbm, out_hbm, idx_v, rows_v, sem):
        wid = lax.axis_index("s") * _NC + lax.axis_index("c")
        base = wid * _BPW
        for c in range(_BPW // _CH):
            pltpu.sync_copy(idx_hbm.at[pl.ds(base + c * _CH, _CH)], idx_v)
            pltpu.async_copy(table_hbm.at[idx_v],
                             rows_v.at[pl.ds(c * _CH, _CH)], sem).wait()
        pltpu.sync_copy(rows_v, out_hbm.at[pl.ds(base, _BPW)])

    return gather_k(table, idx)


def kernel(x, w):
    flat = x.reshape(-1, K)
    idx2d = _argmin_indices(flat, w)
    wt = _transpose_w(w)
    quant = _gather_rows(wt, idx2d.reshape(M))
    return quant.reshape(x.shape)
